# Initial kernel scaffold; baseline (speedup 1.0000x reference)
import jax
import jax.numpy as jnp
from jax.experimental import pallas as pl


def kernel(x, Wr, br, W1, b1, W2, b2):
    raise NotImplementedError("write your pallas kernel here")


# speedup vs baseline: 10.3515x; 10.3515x over previous
import functools

import jax
import jax.numpy as jnp
from jax import lax
from jax.experimental import pallas as pl
from jax.experimental.pallas import tpu as pltpu
from jax.experimental.pallas import tpu_sc as plsc

D_MODEL = 768
D_FF = 3072
N_EXP = 64
N_TOK = 8192
TB = 512
N_TB = N_TOK // TB
B = 128
G = N_TOK // B + N_EXP
NW = 32
ROWS_W = N_TOK // NW
CH = 4
CHROWS = ROWS_W // CH


def _router_body(x_ref, wr_ref, br_ref, eid_ref, rank_ref, cnt_ref, cnt_s):
    i = pl.program_id(0)

    @pl.when(i == 0)
    def _():
        cnt_s[...] = jnp.zeros_like(cnt_s)

    logits = jnp.dot(x_ref[...], wr_ref[...],
                     preferred_element_type=jnp.float32) + br_ref[...]
    m = jnp.max(logits, axis=1, keepdims=True)
    lane = lax.broadcasted_iota(jnp.int32, (TB, N_EXP), 1)
    eid = jnp.min(jnp.where(logits == m, lane, N_EXP), axis=1, keepdims=True)
    onehot = (eid == lane).astype(jnp.float32)
    row = lax.broadcasted_iota(jnp.int32, (TB, TB), 0)
    col = lax.broadcasted_iota(jnp.int32, (TB, TB), 1)
    ltri = (col < row).astype(jnp.float32)
    prev = cnt_s[...]
    cum = jnp.dot(ltri, onehot, preferred_element_type=jnp.float32) + prev
    rank = jnp.sum(cum * onehot, axis=1, keepdims=True)
    total = prev + jnp.sum(onehot, axis=0, keepdims=True)
    cnt_s[...] = total
    eid_ref[...] = eid
    rank_ref[...] = rank.astype(jnp.int32)
    cnt_ref[...] = total.astype(jnp.int32)


_router = pl.pallas_call(
    _router_body,
    grid=(N_TB,),
    in_specs=[
        pl.BlockSpec((TB, D_MODEL), lambda i: (i, 0)),
        pl.BlockSpec((D_MODEL, N_EXP), lambda i: (0, 0)),
        pl.BlockSpec((1, N_EXP), lambda i: (0, 0)),
    ],
    out_specs=[
        pl.BlockSpec((TB, 1), lambda i: (i, 0)),
        pl.BlockSpec((TB, 1), lambda i: (i, 0)),
        pl.BlockSpec((1, N_EXP), lambda i: (0, 0)),
    ],
    out_shape=[
        jax.ShapeDtypeStruct((N_TOK, 1), jnp.int32),
        jax.ShapeDtypeStruct((N_TOK, 1), jnp.int32),
        jax.ShapeDtypeStruct((1, N_EXP), jnp.int32),
    ],
    scratch_shapes=[pltpu.VMEM((1, N_EXP), jnp.float32)],
)


def _slots_body(cnt_ref, eid_ref, rank_ref, p_ref, be_ref, nused_ref):
    i = pl.program_id(0)
    bstart = []
    acc = 0
    for e in range(N_EXP):
        bstart.append(acc)
        acc = acc + (cnt_ref[0, e] + (B - 1)) // B
    nblocks = acc

    eid = eid_ref[...]
    slot_base = jnp.zeros((TB, 1), jnp.int32)
    for e in range(N_EXP):
        slot_base = jnp.where(eid == e, bstart[e] * B, slot_base)
    p_ref[...] = slot_base + rank_ref[...]

    @pl.when(i == 0)
    def _():
        g = lax.broadcasted_iota(jnp.int32, (1, G), 1)
        nge = jnp.zeros((1, G), jnp.int32)
        for e in range(N_EXP):
            nge = nge + (g >= bstart[e]).astype(jnp.int32)
        be_ref[...] = nge - 1
        nused_ref[0, 0] = nblocks


_slots = pl.pallas_call(
    _slots_body,
    grid=(N_TB,),
    in_specs=[
        pl.BlockSpec(memory_space=pltpu.SMEM),
        pl.BlockSpec((TB, 1), lambda i: (i, 0)),
        pl.BlockSpec((TB, 1), lambda i: (i, 0)),
    ],
    out_specs=[
        pl.BlockSpec((TB, 1), lambda i: (i, 0)),
        pl.BlockSpec((1, G), lambda i: (0, 0)),
        pl.BlockSpec(memory_space=pltpu.SMEM),
    ],
    out_shape=[
        jax.ShapeDtypeStruct((N_TOK, 1), jnp.int32),
        jax.ShapeDtypeStruct((1, G), jnp.int32),
        jax.ShapeDtypeStruct((1, 1), jnp.int32),
    ],
)


@functools.cache
def _sc_kernels():
    mesh = plsc.VectorSubcoreMesh(core_axis_name="c", subcore_axis_name="s")
    scratch = [
        pltpu.VMEM((CH, CHROWS), jnp.int32),
        pltpu.VMEM((CHROWS, D_MODEL), jnp.float32),
        pltpu.SemaphoreType.DMA,
    ]

    @functools.partial(
        pl.kernel,
        mesh=mesh,
        out_type=jax.ShapeDtypeStruct((G * B, D_MODEL), jnp.float32),
        scratch_types=scratch,
    )
    def dispatch(x_hbm, p_hbm, xg_hbm, idx_v, buf_v, sem):
        wid = lax.axis_index("s") * 2 + lax.axis_index("c")
        pltpu.sync_copy(p_hbm.at[wid], idx_v)
        for j in range(CH):
            base = wid * ROWS_W + j * CHROWS
            pltpu.sync_copy(x_hbm.at[pl.ds(base, CHROWS)], buf_v)
            pltpu.async_copy(buf_v, xg_hbm.at[idx_v.at[j]], sem).wait()

    @functools.partial(
        pl.kernel,
        mesh=mesh,
        out_type=jax.ShapeDtypeStruct((N_TOK, D_MODEL), jnp.float32),
        scratch_types=scratch,
    )
    def combine(y_hbm, p_hbm, out_hbm, idx_v, buf_v, sem):
        wid = lax.axis_index("s") * 2 + lax.axis_index("c")
        pltpu.sync_copy(p_hbm.at[wid], idx_v)
        for j in range(CH):
            base = wid * ROWS_W + j * CHROWS
            pltpu.async_copy(y_hbm.at[idx_v.at[j]], buf_v, sem).wait()
            pltpu.sync_copy(buf_v, out_hbm.at[pl.ds(base, CHROWS)])

    return dispatch, combine


def _ffn_body(be_ref, nused_ref, xg_ref, w1_ref, b1_ref, w2_ref, b2_ref,
              y_ref):
    i = pl.program_id(0)

    @pl.when(i < nused_ref[0])
    def _():
        h = jnp.dot(xg_ref[...], w1_ref[0],
                    preferred_element_type=jnp.float32) + b1_ref[0]
        h = jnp.maximum(h, 0.0)
        y_ref[...] = jnp.dot(h, w2_ref[0],
                             preferred_element_type=jnp.float32) + b2_ref[0]


_ffn = pl.pallas_call(
    _ffn_body,
    grid_spec=pltpu.PrefetchScalarGridSpec(
        num_scalar_prefetch=2,
        grid=(G,),
        in_specs=[
            pl.BlockSpec((B, D_MODEL), lambda i, be, nu: (i, 0)),
            pl.BlockSpec((1, D_MODEL, D_FF), lambda i, be, nu: (be[i], 0, 0)),
            pl.BlockSpec((1, 1, D_FF), lambda i, be, nu: (be[i], 0, 0)),
            pl.BlockSpec((1, D_FF, D_MODEL), lambda i, be, nu: (be[i], 0, 0)),
            pl.BlockSpec((1, 1, D_MODEL), lambda i, be, nu: (be[i], 0, 0)),
        ],
        out_specs=pl.BlockSpec((B, D_MODEL), lambda i, be, nu: (i, 0)),
    ),
    out_shape=jax.ShapeDtypeStruct((G * B, D_MODEL), jnp.float32),
)


def kernel(x, Wr, br, W1, b1, W2, b2):
    orig_shape = x.shape
    x_flat = x.reshape(-1, D_MODEL)
    eid, rank, counts = _router(x_flat, Wr, br.reshape(1, N_EXP))
    p, be, nused = _slots(counts, eid, rank)
    p3 = p.reshape(NW, CH, CHROWS)
    dispatch, combine = _sc_kernels()
    xg = dispatch(x_flat, p3)
    y = _ffn(be.reshape(G), nused.reshape(1), xg, W1,
             b1.reshape(N_EXP, 1, D_FF), W2, b2.reshape(N_EXP, 1, D_MODEL))
    out = combine(y, p3)
    return out.reshape(orig_shape)
